# baseline (speedup 1.0000x reference)
import dataclasses
import functools

import jax
import jax.numpy as jnp
from jax import lax
from jax.experimental import pallas as pl
from jax.experimental.pallas import tpu as pltpu
from jax.experimental.pallas import tpu_sc as plsc

NW = 32
CHUNK_ROWS = 4
KROWS = 512


def _cp():
    cp = pltpu.CompilerParams()
    if "needs_layout_passes" in pltpu.CompilerParams.__dataclass_fields__:
        cp = dataclasses.replace(cp, needs_layout_passes=False)
    return cp


@functools.partial(jax.jit, static_argnums=(6,))
def _sc_relayout(e_t, i_t, g_t, e_tail, i_tail, g_tail, V):
    D = e_t.shape[0]
    nstep = V // KROWS
    srows = KROWS // 8

    mesh = plsc.VectorSubcoreMesh(core_axis_name="c", subcore_axis_name="s")
    cp = _cp()
    if "use_tc_tiling_on_sc" in pltpu.CompilerParams.__dataclass_fields__:
        cp = dataclasses.replace(cp, use_tc_tiling_on_sc=True)

    out_sd = jax.ShapeDtypeStruct((V * D // 128, 128), jnp.float32)

    @functools.partial(
        pl.kernel,
        out_type=(out_sd, out_sd, out_sd),
        mesh=mesh,
        compiler_params=cp,
        scratch_types=[
            pltpu.VMEM((D, KROWS + 8), jnp.float32),
            pltpu.VMEM((D, KROWS + 8), jnp.float32),
            pltpu.VMEM((srows, 128), jnp.float32),
            pltpu.VMEM((srows, 128), jnp.float32),
            pltpu.VMEM((D, 128), jnp.float32),
            pltpu.SemaphoreType.DMA,
            pltpu.SemaphoreType.DMA,
            pltpu.SemaphoreType.DMA,
            pltpu.SemaphoreType.DMA,
        ],
    )
    def k1(e_hbm, i_hbm, g_hbm, et_hbm, it_hbm, gt_hbm,
           eo_hbm, io_hbm, go_hbm,
           w0, w1, p0, p1, wt, sw0, sw1, sp0, sp1):
        cid = lax.axis_index("c")
        sid = lax.axis_index("s")
        wid = sid * 2 + cid

        iota = lax.iota(jnp.int32, 16)

        def convert(src_hbm, dst_hbm):
            nmine = nstep // NW

            def issue_w(j, wb, sw):
                pltpu.async_copy(src_hbm.at[:, pl.ds(j * KROWS, KROWS)],
                                 wb.at[:, pl.ds(0, KROWS)], sw)

            def wait_w(j, wb, sw):
                pltpu.make_async_copy(src_hbm.at[:, pl.ds(j * KROWS, KROWS)],
                                      wb.at[:, pl.ds(0, KROWS)], sw).wait()

            def transpose(wb, pb):
                @pl.loop(0, srows)
                def _(t):
                    for s in range(8):
                        r = t * 8 + s
                        vec = plsc.load_gather(wb, [iota, jnp.full((16,), 0, jnp.int32) + r])
                        pb[t, pl.ds(s * 16, 16)] = vec

            def wr(j, pb, sp):
                pltpu.async_copy(pb, dst_hbm.at[pl.ds(j * srows, srows)], sp)

            def wr_wait(j, pb, sp):
                pltpu.make_async_copy(pb, dst_hbm.at[pl.ds(j * srows, srows)],
                                      sp).wait()

            issue_w(wid, w0, sw0)

            @pl.loop(0, nmine, step=2)
            def _(m):
                j0 = wid + m * NW
                j1 = wid + (m + 1) * NW

                @pl.when(m + 1 < nmine)
                def _():
                    issue_w(j1, w1, sw1)

                wait_w(j0, w0, sw0)

                @pl.when(m >= 2)
                def _():
                    wr_wait(wid + (m - 2) * NW, p0, sp0)

                transpose(w0, p0)
                wr(j0, p0, sp0)

                @pl.when(m + 1 < nmine)
                def _():
                    @pl.when(m + 2 < nmine)
                    def _():
                        issue_w(wid + (m + 2) * NW, w0, sw0)

                    wait_w(j1, w1, sw1)

                    @pl.when(m >= 2)
                    def _():
                        wr_wait(wid + (m - 1) * NW, p1, sp1)

                    transpose(w1, p1)
                    wr(j1, p1, sp1)

            last_a = nmine - 1 if nmine % 2 else nmine - 2
            last_b = nmine - 2 if nmine % 2 else nmine - 1
            if nmine >= 1:
                wr_wait(wid + last_a * NW, p0, sp0)
            if nmine >= 2:
                wr_wait(wid + last_b * NW, p1, sp1)

        convert(e_hbm, eo_hbm)
        convert(i_hbm, io_hbm)
        convert(g_hbm, go_hbm)

        rem0 = (nstep // NW) * NW
        nrem = nstep - rem0

        for tix, (src_hbm, tl_hbm, dst_hbm) in enumerate(
                ((e_hbm, et_hbm, eo_hbm), (i_hbm, it_hbm, io_hbm),
                 (g_hbm, gt_hbm, go_hbm))):
            @pl.when(jnp.logical_and(wid >= tix * 10, wid < tix * 10 + nrem))
            def _():
                j = rem0 + (wid - tix * 10)
                pltpu.sync_copy(src_hbm.at[:, pl.ds(j * KROWS, KROWS)],
                                w0.at[:, pl.ds(0, KROWS)])

                @pl.loop(0, srows)
                def _(t):
                    for s in range(8):
                        r = t * 8 + s
                        vec = plsc.load_gather(w0, [iota, jnp.full((16,), 0, jnp.int32) + r])
                        p0[t, pl.ds(s * 16, 16)] = vec

                pltpu.sync_copy(p0, dst_hbm.at[pl.ds(j * srows, srows)])

            tail_n = V - nstep * KROWS
            @pl.when(wid == 31 - tix)
            def _():
                pltpu.sync_copy(tl_hbm, wt)
                t0 = (nstep * KROWS) // 8
                ntr = tail_n // 8

                @pl.loop(0, ntr)
                def _(t):
                    for s in range(8):
                        r = (128 - tail_n) + t * 8 + s
                        vec = plsc.load_gather(wt, [iota, jnp.full((16,), 0, jnp.int32) + r])
                        p0[t, pl.ds(s * 16, 16)] = vec

                pltpu.sync_copy(p0.at[pl.ds(0, ntr)],
                                dst_hbm.at[pl.ds(t0, ntr)])

    return k1(e_t, i_t, g_t, e_tail, i_tail, g_tail)


@functools.partial(jax.jit, static_argnums=(4, 5))
def _sc_forward(idx, emb_table, i_emb_table, g_emb_table, B, F):
    D = emb_table.shape[1]
    rows_per_w = B // NW
    chunk_idx = CHUNK_ROWS * F
    nchunk = rows_per_w // CHUNK_ROWS
    groups_per_w = rows_per_w // D

    mesh = plsc.VectorSubcoreMesh(core_axis_name="c", subcore_axis_name="s")
    cp = _cp()
    if "use_tc_tiling_on_sc" in pltpu.CompilerParams.__dataclass_fields__:
        cp = dataclasses.replace(cp, use_tc_tiling_on_sc=False)

    @functools.partial(
        pl.kernel,
        out_type=jax.ShapeDtypeStruct((B // D, D), jnp.float32),
        mesh=mesh,
        compiler_params=cp,
        scratch_types=[
            pltpu.VMEM((nchunk, chunk_idx), jnp.int32),
            pltpu.VMEM((chunk_idx, D), jnp.float32),
            pltpu.VMEM((chunk_idx, D), jnp.float32),
            pltpu.VMEM((chunk_idx, D), jnp.float32),
            pltpu.VMEM((chunk_idx, D), jnp.float32),
            pltpu.VMEM((chunk_idx, D), jnp.float32),
            pltpu.VMEM((chunk_idx, D), jnp.float32),
            pltpu.VMEM((groups_per_w, D), jnp.float32),
            pltpu.SemaphoreType.DMA,
            pltpu.SemaphoreType.DMA,
            pltpu.SemaphoreType.DMA,
            pltpu.SemaphoreType.DMA,
            pltpu.SemaphoreType.DMA,
            pltpu.SemaphoreType.DMA,
        ],
    )
    def sc_kernel(idx_hbm, e_hbm, i_hbm, g_hbm, out_hbm,
                  idx_v, e0, i0, g0, e1, i1, g1, out_v,
                  se0, si0, sg0, se1, si1, sg1):
        cid = lax.axis_index("c")
        sid = lax.axis_index("s")
        wid = sid * 2 + cid

        pltpu.sync_copy(idx_hbm.at[wid], idx_v)

        def issue(j, eb, ib, gb, se, si, sg):
            row = idx_v.at[j]
            pltpu.async_copy(e_hbm.at[row], eb, se)
            pltpu.async_copy(i_hbm.at[row], ib, si)
            pltpu.async_copy(g_hbm.at[row], gb, sg)

        def wait(j, eb, ib, gb, se, si, sg):
            row = idx_v.at[j]
            pltpu.make_async_copy(e_hbm.at[row], eb, se).wait()
            pltpu.make_async_copy(i_hbm.at[row], ib, si).wait()
            pltpu.make_async_copy(g_hbm.at[row], gb, sg).wait()

        chunks_per_group = D // CHUNK_ROWS
        lanes = lax.iota(jnp.int32, 16)

        def compute(j, eb, ib, gb):
            g = j // chunks_per_group
            q = j % chunks_per_group
            vec = out_v[g]
            for r in range(CHUNK_ROWS):
                acc = eb[r * F] * ib[r * F] * gb[r * F]
                for f in range(1, F):
                    k = r * F + f
                    acc = acc + eb[k] * ib[k] * gb[k]
                s = jnp.sum(acc)
                vec = jnp.where(lanes == q * CHUNK_ROWS + r, s, vec)
            out_v[g] = vec

        issue(0, e0, i0, g0, se0, si0, sg0)

        @pl.loop(0, nchunk, step=2)
        def _(j):
            issue(j + 1, e1, i1, g1, se1, si1, sg1)
            wait(j, e0, i0, g0, se0, si0, sg0)
            compute(j, e0, i0, g0)

            @pl.when(j + 2 < nchunk)
            def _():
                issue(j + 2, e0, i0, g0, se0, si0, sg0)

            wait(j + 1, e1, i1, g1, se1, si1, sg1)
            compute(j + 1, e1, i1, g1)

        pltpu.sync_copy(out_v, out_hbm.at[pl.ds(wid * groups_per_w, groups_per_w)])

    return sc_kernel(idx, emb_table, i_emb_table, g_emb_table)


def kernel(sparse_input, emb_table, i_emb_table, g_emb_table):
    B, F = sparse_input.shape
    V, D = emb_table.shape
    rows_per_w = B // NW
    chunk_idx = CHUNK_ROWS * F
    nchunk = rows_per_w // CHUNK_ROWS
    idx = sparse_input.astype(jnp.int32).reshape(NW, nchunk, chunk_idx)
    e_t, i_t, g_t = emb_table.T, i_emb_table.T, g_emb_table.T
    tails = [lax.dynamic_slice(t, (0, V - 128), (D, 128))
             for t in (e_t, i_t, g_t)]
    e_p, i_p, g_p = _sc_relayout(e_t, i_t, g_t, tails[0], tails[1], tails[2], V)
    out = _sc_forward(idx, e_p.reshape(V, D), i_p.reshape(V, D),
                      g_p.reshape(V, D), B, F)
    return out.reshape(B)


# speedup vs baseline: 1.0192x; 1.0192x over previous
import dataclasses
import functools

import jax
import jax.numpy as jnp
from jax import lax
from jax.experimental import pallas as pl
from jax.experimental.pallas import tpu as pltpu
from jax.experimental.pallas import tpu_sc as plsc

NW = 32
CHUNK_ROWS = 4


@functools.partial(jax.jit, static_argnums=(4, 5))
def _sc_forward(idx, emb_table, i_emb_table, g_emb_table, B, F):
    D = emb_table.shape[1]
    rows_per_w = B // NW
    chunk_idx = CHUNK_ROWS * F
    nchunk = rows_per_w // CHUNK_ROWS

    mesh = plsc.VectorSubcoreMesh(core_axis_name="c", subcore_axis_name="s")

    cp = pltpu.CompilerParams()
    if "needs_layout_passes" in pltpu.CompilerParams.__dataclass_fields__:
        cp = dataclasses.replace(cp, needs_layout_passes=False)
    if "use_tc_tiling_on_sc" in pltpu.CompilerParams.__dataclass_fields__:
        cp = dataclasses.replace(cp, use_tc_tiling_on_sc=False)

    groups_per_w = rows_per_w // D

    @functools.partial(
        pl.kernel,
        out_type=jax.ShapeDtypeStruct((B // D, D), jnp.float32),
        mesh=mesh,
        compiler_params=cp,
        scratch_types=[
            pltpu.VMEM((nchunk, chunk_idx), jnp.int32),
            pltpu.VMEM((chunk_idx, D), jnp.float32),
            pltpu.VMEM((chunk_idx, D), jnp.float32),
            pltpu.VMEM((chunk_idx, D), jnp.float32),
            pltpu.VMEM((chunk_idx, D), jnp.float32),
            pltpu.VMEM((chunk_idx, D), jnp.float32),
            pltpu.VMEM((chunk_idx, D), jnp.float32),
            pltpu.VMEM((groups_per_w, D), jnp.float32),
            pltpu.SemaphoreType.DMA,
            pltpu.SemaphoreType.DMA,
            pltpu.SemaphoreType.DMA,
            pltpu.SemaphoreType.DMA,
            pltpu.SemaphoreType.DMA,
            pltpu.SemaphoreType.DMA,
        ],
    )
    def sc_kernel(idx_hbm, e_hbm, i_hbm, g_hbm, out_hbm,
                  idx_v, e0, i0, g0, e1, i1, g1, out_v,
                  se0, si0, sg0, se1, si1, sg1):
        cid = lax.axis_index("c")
        sid = lax.axis_index("s")
        wid = sid * 2 + cid

        pltpu.sync_copy(idx_hbm.at[wid], idx_v)

        def issue(j, eb, ib, gb, se, si, sg):
            row = idx_v.at[j]
            pltpu.async_copy(e_hbm.at[row], eb, se)
            pltpu.async_copy(i_hbm.at[row], ib, si)
            pltpu.async_copy(g_hbm.at[row], gb, sg)

        def wait(j, eb, ib, gb, se, si, sg):
            row = idx_v.at[j]
            pltpu.make_async_copy(e_hbm.at[row], eb, se).wait()
            pltpu.make_async_copy(i_hbm.at[row], ib, si).wait()
            pltpu.make_async_copy(g_hbm.at[row], gb, sg).wait()

        chunks_per_group = D // CHUNK_ROWS
        lanes = lax.iota(jnp.int32, 16)

        def compute(j, eb, ib, gb):
            g = j // chunks_per_group
            q = j % chunks_per_group
            vec = out_v[g]
            for r in range(CHUNK_ROWS):
                acc = eb[r * F] * ib[r * F] * gb[r * F]
                for f in range(1, F):
                    k = r * F + f
                    acc = acc + eb[k] * ib[k] * gb[k]
                s = jnp.sum(acc)
                vec = jnp.where(lanes == q * CHUNK_ROWS + r, s, vec)
            out_v[g] = vec

        issue(0, e0, i0, g0, se0, si0, sg0)

        @pl.loop(0, nchunk, step=2)
        def _(j):
            issue(j + 1, e1, i1, g1, se1, si1, sg1)
            wait(j, e0, i0, g0, se0, si0, sg0)
            compute(j, e0, i0, g0)

            @pl.when(j + 2 < nchunk)
            def _():
                issue(j + 2, e0, i0, g0, se0, si0, sg0)

            wait(j + 1, e1, i1, g1, se1, si1, sg1)
            compute(j + 1, e1, i1, g1)

        pltpu.sync_copy(out_v, out_hbm.at[pl.ds(wid * groups_per_w, groups_per_w)])

    return sc_kernel(idx, emb_table, i_emb_table, g_emb_table)


def _row_major(table):
    V, D = table.shape
    packed = table.reshape(V * D // 128, 128)
    packed = jax.lax.optimization_barrier(packed)
    return packed.reshape(V, D)


def kernel(sparse_input, emb_table, i_emb_table, g_emb_table):
    B, F = sparse_input.shape
    rows_per_w = B // NW
    chunk_idx = CHUNK_ROWS * F
    nchunk = rows_per_w // CHUNK_ROWS
    idx = sparse_input.astype(jnp.int32).reshape(NW, nchunk, chunk_idx)
    out = _sc_forward(idx, _row_major(emb_table), _row_major(i_emb_table),
                      _row_major(g_emb_table), B, F)
    return out.reshape(B)
